# Initial kernel scaffold; baseline (speedup 1.0000x reference)
import jax
import jax.numpy as jnp
from jax.experimental import pallas as pl


def kernel(x, table):
    raise NotImplementedError("write your pallas kernel here")


# speedup vs baseline: 4.6697x; 4.6697x over previous
import functools

import jax
import jax.numpy as jnp
from jax import lax
from jax.experimental import pallas as pl
from jax.experimental.pallas import tpu as pltpu, tpu_sc as plsc

HID = 64
BATCH = 16384
HIST = 200

NC = 2
NS = 16
NW = NC * NS

CHUNK = 128
K = 4
BLOCK = K * CHUNK

B = BATCH * HIST
PER_W = B // NW
NB = PER_W // BLOCK


def _embed_sc(x_flat, table):
    mesh = plsc.VectorSubcoreMesh(core_axis_name="c", subcore_axis_name="s")

    @functools.partial(
        pl.kernel,
        out_type=jax.ShapeDtypeStruct((B, HID), jnp.float32),
        mesh=mesh,
        scratch_types=[
            pltpu.VMEM((K, CHUNK), jnp.int32),
            pltpu.VMEM((BLOCK, HID), jnp.float32),
            pltpu.SemaphoreType.DMA,
        ],
        compiler_params=pltpu.CompilerParams(use_tc_tiling_on_sc=False),
    )
    def k(idx_hbm, table_hbm, out_hbm, idx_v, rows_v, sem):
        wid = lax.axis_index("s") * NC + lax.axis_index("c")

        @pl.loop(0, NB)
        def _body(b):
            pltpu.sync_copy(idx_hbm.at[wid, b], idx_v)
            for j in range(K):
                pltpu.async_copy(
                    table_hbm.at[idx_v.at[j]],
                    rows_v.at[pl.ds(j * CHUNK, CHUNK)],
                    sem,
                )
            for j in range(K):
                pltpu.make_async_copy(
                    table_hbm.at[idx_v.at[j]],
                    rows_v.at[pl.ds(j * CHUNK, CHUNK)],
                    sem,
                ).wait()
            base = (wid * NB + b) * BLOCK
            pltpu.sync_copy(rows_v, out_hbm.at[pl.ds(base, BLOCK)])

    return k(x_flat, table)


def kernel(x, table):
    x_flat = x.reshape(NW, NB, K, CHUNK)
    out = _embed_sc(x_flat, table)
    return out.reshape(BATCH, HIST, HID)


# speedup vs baseline: 4.9714x; 1.0646x over previous
import functools

import jax
import jax.numpy as jnp
from jax import lax
from jax.experimental import pallas as pl
from jax.experimental.pallas import tpu as pltpu, tpu_sc as plsc

HID = 64
BATCH = 16384
HIST = 200

NC = 2
NS = 16
NW = NC * NS

CHUNK = 128
K = 4
BLOCK = K * CHUNK

B = BATCH * HIST
PER_W = B // NW
NB = PER_W // BLOCK

NIB = 4
NRB = 2


def _embed_sc(x_flat, table):
    mesh = plsc.VectorSubcoreMesh(core_axis_name="c", subcore_axis_name="s")

    @functools.partial(
        pl.kernel,
        out_type=jax.ShapeDtypeStruct((B, HID), jnp.float32),
        mesh=mesh,
        scratch_types=[
            pltpu.VMEM((NIB, K, CHUNK), jnp.int32),
            pltpu.VMEM((NRB, BLOCK, HID), jnp.float32),
        ]
        + [pltpu.SemaphoreType.DMA] * (NIB + 2 * NRB),
        compiler_params=pltpu.CompilerParams(use_tc_tiling_on_sc=False),
    )
    def k(idx_hbm, table_hbm, out_hbm, idx_v, rows_v, *sems):
        isem = sems[:NIB]
        gsem = sems[NIB:NIB + NRB]
        osem = sems[NIB + NRB:]
        wid = lax.axis_index("s") * NC + lax.axis_index("c")

        def wait_rows(rb):
            pltpu.make_async_copy(
                out_hbm.at[pl.ds(0, BLOCK)], rows_v.at[rb], gsem[rb]
            ).wait()

        for ib in range(NIB):
            pltpu.async_copy(idx_hbm.at[wid, ib], idx_v.at[ib], isem[ib])

        @pl.loop(0, NB, step=NIB)
        def _body(s):
            for off in range(NIB):
                b = s + off
                rb = off % NRB
                ib = off
                if off >= NRB:
                    pltpu.make_async_copy(
                        rows_v.at[rb], out_hbm.at[pl.ds(0, BLOCK)], osem[rb]
                    ).wait()
                else:
                    @pl.when(s > 0)
                    def _():
                        pltpu.make_async_copy(
                            rows_v.at[rb], out_hbm.at[pl.ds(0, BLOCK)],
                            osem[rb],
                        ).wait()
                pltpu.make_async_copy(
                    idx_hbm.at[wid, 0], idx_v.at[ib], isem[ib]
                ).wait()
                for j in range(K):
                    pltpu.async_copy(
                        table_hbm.at[idx_v.at[ib, j]],
                        rows_v.at[rb, pl.ds(j * CHUNK, CHUNK)],
                        gsem[rb],
                    )
                wait_rows(rb)
                @pl.when(b + NIB < NB)
                def _():
                    pltpu.async_copy(
                        idx_hbm.at[wid, b + NIB], idx_v.at[ib], isem[ib]
                    )
                base = (wid * NB + b) * BLOCK
                pltpu.async_copy(
                    rows_v.at[rb], out_hbm.at[pl.ds(base, BLOCK)], osem[rb]
                )

        for rb in range(NRB):
            pltpu.make_async_copy(
                rows_v.at[rb], out_hbm.at[pl.ds(0, BLOCK)], osem[rb]
            ).wait()

    return k(x_flat, table)


def kernel(x, table):
    x_flat = x.reshape(NW, NB, K, CHUNK)
    out = _embed_sc(x_flat, table)
    return out.reshape(BATCH, HIST, HID)


# speedup vs baseline: 5.8170x; 1.1701x over previous
import functools

import jax
import jax.numpy as jnp
from jax import lax
from jax.experimental import pallas as pl
from jax.experimental.pallas import tpu as pltpu, tpu_sc as plsc

HID = 64
BATCH = 16384
HIST = 200

NC = 2
NS = 16
NW = NC * NS

CHUNK = 128
K = 4
BLOCK = K * CHUNK

B = BATCH * HIST
PER_W = B // NW
NB = PER_W // BLOCK

NIB = 4
NRB = 2


def _embed_sc(x_flat, table):
    mesh = plsc.VectorSubcoreMesh(core_axis_name="c", subcore_axis_name="s")

    @functools.partial(
        pl.kernel,
        out_type=jax.ShapeDtypeStruct((B, HID), jnp.float32),
        mesh=mesh,
        scratch_types=[
            pltpu.VMEM((NIB, K, CHUNK), jnp.int32),
            pltpu.VMEM((NRB, BLOCK, HID), jnp.float32),
            pltpu.VMEM_SHARED((5000, HID), jnp.float32),
        ]
        + [pltpu.SemaphoreType.DMA] * (NIB + 2 * NRB),
        compiler_params=pltpu.CompilerParams(use_tc_tiling_on_sc=False),
    )
    def k(idx_hbm, table_hbm, out_hbm, idx_v, rows_v, table_sh, *sems):
        isem = sems[:NIB]
        gsem = sems[NIB:NIB + NRB]
        osem = sems[NIB + NRB:]
        sid = lax.axis_index("s")
        wid = sid * NC + lax.axis_index("c")

        @pl.when(sid == 0)
        def _():
            pltpu.sync_copy(table_hbm, table_sh)
        plsc.subcore_barrier()

        def wait_rows(rb):
            pltpu.make_async_copy(
                out_hbm.at[pl.ds(0, BLOCK)], rows_v.at[rb], gsem[rb]
            ).wait()

        for ib in range(NIB):
            pltpu.async_copy(idx_hbm.at[wid, ib], idx_v.at[ib], isem[ib])

        @pl.loop(0, NB, step=NIB)
        def _body(s):
            for off in range(NIB):
                b = s + off
                rb = off % NRB
                ib = off
                if off >= NRB:
                    pltpu.make_async_copy(
                        rows_v.at[rb], out_hbm.at[pl.ds(0, BLOCK)], osem[rb]
                    ).wait()
                else:
                    @pl.when(s > 0)
                    def _():
                        pltpu.make_async_copy(
                            rows_v.at[rb], out_hbm.at[pl.ds(0, BLOCK)],
                            osem[rb],
                        ).wait()
                pltpu.make_async_copy(
                    idx_hbm.at[wid, 0], idx_v.at[ib], isem[ib]
                ).wait()
                for j in range(K):
                    pltpu.async_copy(
                        table_sh.at[idx_v.at[ib, j]],
                        rows_v.at[rb, pl.ds(j * CHUNK, CHUNK)],
                        gsem[rb],
                    )
                wait_rows(rb)
                @pl.when(b + NIB < NB)
                def _():
                    pltpu.async_copy(
                        idx_hbm.at[wid, b + NIB], idx_v.at[ib], isem[ib]
                    )
                base = (wid * NB + b) * BLOCK
                pltpu.async_copy(
                    rows_v.at[rb], out_hbm.at[pl.ds(base, BLOCK)], osem[rb]
                )

        for rb in range(NRB):
            pltpu.make_async_copy(
                rows_v.at[rb], out_hbm.at[pl.ds(0, BLOCK)], osem[rb]
            ).wait()

    return k(x_flat, table)


def kernel(x, table):
    x_flat = x.reshape(NW, NB, K, CHUNK)
    out = _embed_sc(x_flat, table)
    return out.reshape(BATCH, HIST, HID)


# speedup vs baseline: 5.8218x; 1.0008x over previous
import functools

import jax
import jax.numpy as jnp
from jax import lax
from jax.experimental import pallas as pl
from jax.experimental.pallas import tpu as pltpu, tpu_sc as plsc

HID = 64
BATCH = 16384
HIST = 200
NROW = 5000

NC = 2
NS = 16
NW = NC * NS

R = 2
ROWS_W = BATCH // NW
NB = ROWS_W // R
SPLITS = ((0, 128), (128, 72))

NIB = 4
NRB = 2


def _embed_sc(x_blk, table):
    mesh = plsc.VectorSubcoreMesh(core_axis_name="c", subcore_axis_name="s")

    @functools.partial(
        pl.kernel,
        out_type=jax.ShapeDtypeStruct((BATCH, HIST, HID), jnp.float32),
        mesh=mesh,
        scratch_types=[
            pltpu.VMEM((NIB, R, HIST), jnp.int32),
            pltpu.VMEM((NRB, R, HIST, HID), jnp.float32),
            pltpu.VMEM_SHARED((NROW, HID), jnp.float32),
        ]
        + [pltpu.SemaphoreType.DMA] * (NIB + 2 * NRB),
        compiler_params=pltpu.CompilerParams(use_tc_tiling_on_sc=False),
    )
    def k(idx_hbm, table_hbm, out_hbm, idx_v, rows_v, table_sh, *sems):
        isem = sems[:NIB]
        gsem = sems[NIB:NIB + NRB]
        osem = sems[NIB + NRB:]
        sid = lax.axis_index("s")
        wid = sid * NC + lax.axis_index("c")

        @pl.when(sid == 0)
        def _():
            pltpu.sync_copy(table_hbm, table_sh)
        plsc.subcore_barrier()

        def drain_gathers(rb):
            pltpu.make_async_copy(
                out_hbm.at[pl.ds(0, R)], rows_v.at[rb], gsem[rb]
            ).wait()

        def drain_store(rb):
            pltpu.make_async_copy(
                rows_v.at[rb], out_hbm.at[pl.ds(0, R)], osem[rb]
            ).wait()

        for ib in range(NIB):
            pltpu.async_copy(idx_hbm.at[wid, ib], idx_v.at[ib], isem[ib])

        @pl.loop(0, NB, step=NIB)
        def _body(s):
            for off in range(NIB):
                b = s + off
                rb = off % NRB
                ib = off
                if off >= NRB:
                    drain_store(rb)
                else:
                    @pl.when(s > 0)
                    def _():
                        drain_store(rb)
                pltpu.make_async_copy(
                    idx_hbm.at[wid, 0], idx_v.at[ib], isem[ib]
                ).wait()
                for r in range(R):
                    for c0, cn in SPLITS:
                        pltpu.async_copy(
                            table_sh.at[idx_v.at[ib, r, pl.ds(c0, cn)]],
                            rows_v.at[rb, r, pl.ds(c0, cn)],
                            gsem[rb],
                        )
                drain_gathers(rb)
                @pl.when(b + NIB < NB)
                def _():
                    pltpu.async_copy(
                        idx_hbm.at[wid, b + NIB], idx_v.at[ib], isem[ib]
                    )
                pltpu.async_copy(
                    rows_v.at[rb],
                    out_hbm.at[pl.ds(wid * ROWS_W + b * R, R)],
                    osem[rb],
                )

        for rb in range(NRB):
            drain_store(rb)

    return k(x_blk, table)


def kernel(x, table):
    x_blk = x.reshape(NW, NB, R, HIST)
    return _embed_sc(x_blk, table)
